# Initial kernel scaffold; baseline (speedup 1.0000x reference)
import jax
import jax.numpy as jnp
from jax.experimental import pallas as pl


def kernel(pre, tar):
    raise NotImplementedError("write your pallas kernel here")


# speedup vs baseline: 1.8183x; 1.8183x over previous
import functools

import jax
import jax.numpy as jnp
from jax import lax
from jax.experimental import pallas as pl
from jax.experimental.pallas import tpu as pltpu
from jax.experimental.pallas import tpu_sc as plsc

B = 16384

try:
    _info = plsc.get_sparse_core_info()
    NC, NS, L = _info.num_cores, _info.num_subcores, _info.num_lanes
except Exception:
    NC, NS, L = 2, 16, 16
NW = NC * NS
CPW = B // NW
NCHUNK = CPW // L

_ONE = jnp.float32(1.0)
_ZERO = jnp.float32(0.0)
_ERR = jnp.float32(0.09)
_THR2 = jnp.float32(0.03)


def _ranks(v):
    r = [jnp.zeros((L,), jnp.int32) for _ in range(4)]
    one = jnp.ones((L,), jnp.int32)
    zero = jnp.zeros((L,), jnp.int32)
    for a in range(4):
        for b in range(a + 1, 4):
            cmp = v[a] <= v[b]
            r[b] = r[b] + jnp.where(cmp, one, zero)
            r[a] = r[a] + jnp.where(cmp, zero, one)
    return r


def _group_body(p, t):
    m = jnp.maximum(jnp.maximum(p[0], p[1]), jnp.maximum(p[2], p[3]))
    u = [jnp.exp(pk - m) for pk in p]
    s = u[0] + u[1] + u[2] + u[3]
    o = [jnp.float32(3.0) * (uk / s) for uk in u]

    close4 = ((jnp.abs(o[0] - t[0]) <= _ERR) & (jnp.abs(o[1] - t[1]) <= _ERR)
              & (jnp.abs(o[2] - t[2]) <= _ERR) & (jnp.abs(o[3] - t[3]) <= _ERR))

    a0 = jnp.minimum(t[0], t[1]); a1 = jnp.maximum(t[0], t[1])
    b2 = jnp.minimum(t[2], t[3]); b3 = jnp.maximum(t[2], t[3])
    s0 = jnp.minimum(a0, b2); c2 = jnp.maximum(a0, b2)
    c1 = jnp.minimum(a1, b3)
    s1 = jnp.minimum(c1, c2); s2 = jnp.maximum(c1, c2)
    diff1 = jnp.abs(s0 - s1)
    diff2 = jnp.abs(s2 - s1)

    rt = _ranks(t)
    ro = _ranks(o)

    cond_a = (diff1 < _THR2) & (diff2 < _THR2)
    cond_b = (diff1 < _THR2) & (ro[2] == rt[2])
    cond_c = (diff2 < _THR2) & (ro[0] == rt[0])
    jump = close4 & (cond_a | cond_b | cond_c)

    iz = [jnp.where(t[k] == _ZERO, 1, 0).astype(jnp.int32) for k in range(4)]
    judge0 = iz[0] + iz[1] + iz[2] + iz[3]
    eq_all = ((ro[0] == rt[0]) & (ro[1] == rt[1])
              & (ro[2] == rt[2]) & (ro[3] == rt[3]))

    def tag_eq(k):
        acc = (ro[0] == k) & (rt[0] == k)
        for i in range(1, 4):
            acc = acc | ((ro[i] == k) & (rt[i] == k))
        return acc

    cond_j2 = tag_eq(2) & tag_eq(3)
    cond_j3 = ro[3] == 3

    one = jnp.full((L,), _ONE)
    zero = jnp.full((L,), _ZERO)
    branch23 = jnp.where(judge0 == 2, jnp.where(cond_j2, one, zero),
                         jnp.where(judge0 == 3, jnp.where(cond_j3, one, zero), zero))
    calc = jnp.where(jump, one, jnp.where(judge0 < 2, one, branch23))
    acc = jnp.where(jump, one,
                    jnp.where(judge0 < 2, jnp.where(eq_all, one, zero), branch23))
    return calc, acc


def _make_kernel():
    mesh = plsc.VectorSubcoreMesh(core_axis_name="c", subcore_axis_name="s")

    @functools.partial(
        pl.kernel,
        mesh=mesh,
        out_type=jax.ShapeDtypeStruct((NW, 2, L), jnp.float32),
        scratch_types=[
            pltpu.VMEM((24, CPW), jnp.float32),
            pltpu.VMEM((24, CPW), jnp.float32),
            pltpu.VMEM((2, L), jnp.float32),
        ],
    )
    def recall_kernel(pre_hbm, tar_hbm, out_hbm, pre_v, tar_v, acc_v):
        wid = lax.axis_index("s") * NC + lax.axis_index("c")
        pltpu.sync_copy(pre_hbm.at[wid], pre_v)
        pltpu.sync_copy(tar_hbm.at[wid], tar_v)

        def chunk(ci, carry):
            contrib_acc, valid_acc = carry
            base = pl.multiple_of(ci * L, L)
            calc_num = jnp.zeros((L,), jnp.float32)
            acc_sum = jnp.zeros((L,), jnp.float32)
            for g in range(6):
                p = [pre_v[4 * g + k, pl.ds(base, L)] for k in range(4)]
                t = [tar_v[4 * g + k, pl.ds(base, L)] for k in range(4)]
                calc, acc = _group_body(p, t)
                calc_num = calc_num + calc
                acc_sum = acc_sum + acc
            one = jnp.full((L,), _ONE)
            zero = jnp.full((L,), _ZERO)
            nz = calc_num != zero
            contrib = jnp.where(nz, acc_sum / jnp.maximum(calc_num, one), zero)
            valid = jnp.where(nz, one, zero)
            return contrib_acc + contrib, valid_acc + valid

        contrib_acc, valid_acc = lax.fori_loop(
            0, NCHUNK, chunk,
            (jnp.zeros((L,), jnp.float32), jnp.zeros((L,), jnp.float32)))
        acc_v[0, :] = contrib_acc
        acc_v[1, :] = valid_acc
        pltpu.sync_copy(acc_v, out_hbm.at[wid])

    return recall_kernel


_recall_kernel = _make_kernel()


def kernel(pre, tar):
    pre_r = pre[:, :24].T.reshape(24, NW, CPW).transpose(1, 0, 2)
    tar_r = tar[:, :24].T.reshape(24, NW, CPW).transpose(1, 0, 2)
    parts = _recall_kernel(pre_r, tar_r)
    contrib_total = jnp.sum(parts[:, 0, :])
    valid_total = jnp.sum(parts[:, 1, :])
    return jnp.where(valid_total == _ZERO, _ZERO,
                     contrib_total / jnp.maximum(valid_total, _ONE))


# speedup vs baseline: 1.8654x; 1.0259x over previous
import functools

import jax
import jax.numpy as jnp
from jax import lax
from jax.experimental import pallas as pl
from jax.experimental.pallas import tpu as pltpu
from jax.experimental.pallas import tpu_sc as plsc

B = 16384

try:
    _info = plsc.get_sparse_core_info()
    NC, NS, L = _info.num_cores, _info.num_subcores, _info.num_lanes
except Exception:
    NC, NS, L = 2, 16, 16
NW = NC * NS
CPW = B // NW
NCHUNK = CPW // L

_ONE = jnp.float32(1.0)
_ZERO = jnp.float32(0.0)
_ERR = jnp.float32(0.09)
_THR2 = jnp.float32(0.03)


def _ranks(v):
    r = [jnp.zeros((L,), jnp.int32) for _ in range(4)]
    one = jnp.ones((L,), jnp.int32)
    zero = jnp.zeros((L,), jnp.int32)
    for a in range(4):
        for b in range(a + 1, 4):
            cmp = v[a] <= v[b]
            r[b] = r[b] + jnp.where(cmp, one, zero)
            r[a] = r[a] + jnp.where(cmp, zero, one)
    return r


def _group_body(p, t):
    m = jnp.maximum(jnp.maximum(p[0], p[1]), jnp.maximum(p[2], p[3]))
    u = [jnp.exp(pk - m) for pk in p]
    s = u[0] + u[1] + u[2] + u[3]
    r3 = jnp.float32(3.0) / s
    o = [uk * r3 for uk in u]

    close4 = ((jnp.abs(o[0] - t[0]) <= _ERR) & (jnp.abs(o[1] - t[1]) <= _ERR)
              & (jnp.abs(o[2] - t[2]) <= _ERR) & (jnp.abs(o[3] - t[3]) <= _ERR))

    a0 = jnp.minimum(t[0], t[1]); a1 = jnp.maximum(t[0], t[1])
    b2 = jnp.minimum(t[2], t[3]); b3 = jnp.maximum(t[2], t[3])
    s0 = jnp.minimum(a0, b2); c2 = jnp.maximum(a0, b2)
    c1 = jnp.minimum(a1, b3)
    s1 = jnp.minimum(c1, c2); s2 = jnp.maximum(c1, c2)
    diff1 = jnp.abs(s0 - s1)
    diff2 = jnp.abs(s2 - s1)

    rt = _ranks(t)
    ro = _ranks(o)

    cond_a = (diff1 < _THR2) & (diff2 < _THR2)
    cond_b = (diff1 < _THR2) & (ro[2] == rt[2])
    cond_c = (diff2 < _THR2) & (ro[0] == rt[0])
    jump = close4 & (cond_a | cond_b | cond_c)

    ione = jnp.ones((L,), jnp.int32)
    izero = jnp.zeros((L,), jnp.int32)
    iz = [jnp.where(t[k] == _ZERO, ione, izero) for k in range(4)]
    judge0 = iz[0] + iz[1] + iz[2] + iz[3]
    eq_all = (ro[0] == rt[0]) & (ro[2] == rt[2]) & (ro[3] == rt[3])

    def tag_eq(k):
        acc = (ro[0] == k) & (rt[0] == k)
        for i in range(1, 4):
            acc = acc | ((ro[i] == k) & (rt[i] == k))
        return acc

    cond_j2 = tag_eq(2) & tag_eq(3)
    cond_j3 = ro[3] == 3

    one = jnp.full((L,), _ONE)
    zero = jnp.full((L,), _ZERO)
    branch23 = jnp.where(judge0 == 2, jnp.where(cond_j2, one, zero),
                         jnp.where(judge0 == 3, jnp.where(cond_j3, one, zero), zero))
    calc = jnp.where(jump, one, jnp.where(judge0 < 2, one, branch23))
    acc = jnp.where(jump, one,
                    jnp.where(judge0 < 2, jnp.where(eq_all, one, zero), branch23))
    return calc, acc


def _make_kernel():
    mesh = plsc.VectorSubcoreMesh(core_axis_name="c", subcore_axis_name="s")

    @functools.partial(
        pl.kernel,
        mesh=mesh,
        out_type=jax.ShapeDtypeStruct((NW, 2, L), jnp.float32),
        scratch_types=[
            pltpu.VMEM((24, CPW), jnp.float32),
            pltpu.VMEM((24, CPW), jnp.float32),
            pltpu.VMEM((2, L), jnp.float32),
            pltpu.SemaphoreType.DMA,
            pltpu.SemaphoreType.DMA,
        ],
    )
    def recall_kernel(pre_hbm, tar_hbm, out_hbm, pre_v, tar_v, acc_v,
                      sem_p, sem_t):
        wid = lax.axis_index("s") * NC + lax.axis_index("c")
        cp_p = pltpu.async_copy(pre_hbm.at[wid], pre_v, sem_p)
        cp_t = pltpu.async_copy(tar_hbm.at[wid], tar_v, sem_t)
        cp_p.wait()
        cp_t.wait()

        def chunk(ci, carry):
            contrib_acc, valid_acc = carry
            base = pl.multiple_of(ci * L, L)
            calc_num = jnp.zeros((L,), jnp.float32)
            acc_sum = jnp.zeros((L,), jnp.float32)
            for g in range(6):
                p = [pre_v[4 * g + k, pl.ds(base, L)] for k in range(4)]
                t = [tar_v[4 * g + k, pl.ds(base, L)] for k in range(4)]
                calc, acc = _group_body(p, t)
                calc_num = calc_num + calc
                acc_sum = acc_sum + acc
            one = jnp.full((L,), _ONE)
            zero = jnp.full((L,), _ZERO)
            nz = calc_num != zero
            contrib = jnp.where(nz, acc_sum / jnp.maximum(calc_num, one), zero)
            valid = jnp.where(nz, one, zero)
            return contrib_acc + contrib, valid_acc + valid

        contrib_acc, valid_acc = lax.fori_loop(
            0, NCHUNK, chunk,
            (jnp.zeros((L,), jnp.float32), jnp.zeros((L,), jnp.float32)))
        acc_v[0, :] = contrib_acc
        acc_v[1, :] = valid_acc
        pltpu.sync_copy(acc_v, out_hbm.at[wid])

    return recall_kernel


_recall_kernel = _make_kernel()


def kernel(pre, tar):
    pre_r = pre[:, :24].T.reshape(24, NW, CPW).transpose(1, 0, 2)
    tar_r = tar[:, :24].T.reshape(24, NW, CPW).transpose(1, 0, 2)
    parts = _recall_kernel(pre_r, tar_r)
    contrib_total = jnp.sum(parts[:, 0, :])
    valid_total = jnp.sum(parts[:, 1, :])
    return jnp.where(valid_total == _ZERO, _ZERO,
                     contrib_total / jnp.maximum(valid_total, _ONE))
